# baseline (speedup 1.0000x reference)
import numpy as np
import jax
import jax.numpy as jnp
from jax import lax
from jax.experimental import pallas as pl
from jax.experimental.pallas import tpu as pltpu
from jax.experimental.pallas import tpu_sc as plsc

MAXLEN_ = 2048
D_MODEL_ = 768
BATCH_ = 4
LANES_ = 16

NW_ = 32
ROWS_ = BATCH_ * MAXLEN_
PER_W_ = ROWS_ // NW_
CHUNK_ = 64
NCHUNK_ = PER_W_ // CHUNK_
VECS_ = D_MODEL_ // LANES_


def _positional_encoding(maxlen, d_model):
    pos = np.arange(maxlen, dtype=np.float32)[:, None]
    i = np.arange(d_model, dtype=np.float32)[None, :]
    angle_rates = 1.0 / np.power(10000.0, (2.0 * np.floor(i / 2.0)) / np.float32(d_model))
    angle_rads = pos * angle_rates
    pe = np.zeros((maxlen, d_model), dtype=np.float32)
    pe[:, 0::2] = np.sin(angle_rads[:, 0::2])
    pe[:, 1::2] = np.cos(angle_rads[:, 1::2])
    return jnp.asarray(pe)


def _emb_body(table_hbm, idx_hbm, pe_hbm, out_hbm, idx_v, rows_v, pe_v, sem):
    wid = lax.axis_index("s") * 2 + lax.axis_index("c")
    base = wid * PER_W_
    pbase = lax.rem(base, MAXLEN_)
    pltpu.sync_copy(idx_hbm.at[wid], idx_v)
    for c in range(NCHUNK_):
        gcopy = pltpu.async_copy(table_hbm.at[idx_v.at[c]], rows_v, sem)
        pltpu.sync_copy(pe_hbm.at[pl.ds(pbase + c * CHUNK_, CHUNK_)], pe_v)
        gcopy.wait()

        def add_row(r, carry):
            for j in range(VECS_):
                sl = pl.ds(j * LANES_, LANES_)
                rows_v[r, sl] = rows_v[r, sl] + pe_v[r, sl]
            return carry

        lax.fori_loop(0, CHUNK_, add_row, 0)
        pltpu.sync_copy(rows_v, out_hbm.at[pl.ds(base + c * CHUNK_, CHUNK_)])


def kernel(x, token_emb_table):
    idx = x.reshape(NW_, NCHUNK_, CHUNK_).astype(jnp.int32)
    pe = _positional_encoding(MAXLEN_, D_MODEL_)
    mesh = plsc.VectorSubcoreMesh(core_axis_name="c", subcore_axis_name="s")
    out = pl.kernel(
        _emb_body,
        out_type=jax.ShapeDtypeStruct((ROWS_, D_MODEL_), jnp.float32),
        mesh=mesh,
        scratch_types=[
            pltpu.VMEM((NCHUNK_, CHUNK_), jnp.int32),
            pltpu.VMEM((CHUNK_, D_MODEL_), jnp.float32),
            pltpu.VMEM((CHUNK_, D_MODEL_), jnp.float32),
            pltpu.SemaphoreType.DMA,
        ],
    )(token_emb_table, idx, pe)
    return out.reshape(BATCH_, MAXLEN_, D_MODEL_)
